# Initial kernel scaffold; baseline (speedup 1.0000x reference)
import jax
import jax.numpy as jnp
from jax.experimental import pallas as pl


def kernel(inputs, emb_w):
    raise NotImplementedError("write your pallas kernel here")


# speedup vs baseline: 8.0470x; 8.0470x over previous
import functools

import jax
import jax.numpy as jnp
from jax import lax
from jax.experimental import pallas as pl
from jax.experimental.pallas import tpu as pltpu
from jax.experimental.pallas import tpu_sc as plsc

COMMITMENT_COST = 0.25
K = 8192
D = 32
N = 8192
BT = 128
G = N // BT

NC = 2
NS = 16
NW = NC * NS
BW = N // NW


def _vq_tc_body(x_ref, et_ref, xsq_ref, esq_ref, enc_ref, idx_ref, ppl_ref,
                cnt_ref):
    i = pl.program_id(0)
    mm = jnp.dot(x_ref[...], et_ref[...], preferred_element_type=jnp.float32)
    dist = (xsq_ref[...] + esq_ref[...]) - 2.0 * mm
    H = K // 2
    d0 = dist[:, :H]
    d1 = dist[:, H:]
    m0 = jnp.min(d0, axis=1, keepdims=True)
    m1 = jnp.min(d1, axis=1, keepdims=True)
    half_iota = lax.broadcasted_iota(jnp.int32, (BT, H), 1)
    i0 = jnp.min(jnp.where(d0 == m0, half_iota, jnp.int32(K)), axis=1)
    i1 = jnp.min(jnp.where(d1 == m1, half_iota, jnp.int32(K)), axis=1) + H
    m0b = m0.astype(jnp.bfloat16).astype(jnp.float32)
    idx = jnp.where(m1[:, 0] < m0b[:, 0], i1, i0)
    iota = lax.broadcasted_iota(jnp.int32, (BT, K), 1)
    enc = (iota == idx[:, None]).astype(jnp.float32)
    enc_ref[...] = enc
    idx_ref[...] = idx.reshape(1, 1, BT)

    @pl.when(i == 0)
    def _init():
        cnt_ref[...] = jnp.zeros_like(cnt_ref)

    cnt_ref[...] += jnp.sum(enc, axis=0, keepdims=True)

    @pl.when(i == G - 1)
    def _fini():
        p = cnt_ref[...] * (1.0 / N)
        ent = -jnp.sum(p * jnp.log(p + 1e-10))
        ppl_ref[...] = jnp.exp(ent).reshape(1, 1)


def _vq_tc(flat, et, xsq, esq):
    return pl.pallas_call(
        _vq_tc_body,
        grid=(G,),
        in_specs=[
            pl.BlockSpec((BT, D), lambda i: (i, 0)),
            pl.BlockSpec((D, K), lambda i: (0, 0)),
            pl.BlockSpec((BT, 1), lambda i: (i, 0)),
            pl.BlockSpec((1, K), lambda i: (0, 0)),
        ],
        out_specs=[
            pl.BlockSpec((BT, K), lambda i: (i, 0)),
            pl.BlockSpec((1, 1, BT), lambda i: (i, 0, 0)),
            pl.BlockSpec((1, 1), lambda i: (0, 0)),
        ],
        out_shape=[
            jax.ShapeDtypeStruct((N, K), jnp.float32),
            jax.ShapeDtypeStruct((G, 1, BT), jnp.int32),
            jax.ShapeDtypeStruct((1, 1), jnp.float32),
        ],
        scratch_shapes=[pltpu.VMEM((1, K), jnp.float32)],
    )(flat, et, xsq, esq)


def _sc_gather_body(idx_hbm, x_hbm, emb_hbm, q_out, loss_out,
                    idx_v, rows_v, x_v, st_v, acc_v, sem):
    wid = lax.axis_index("s") * NC + lax.axis_index("c")
    base = wid * BW
    pltpu.sync_copy(idx_hbm.at[wid], idx_v)
    pltpu.sync_copy(x_hbm.at[pl.ds(base, BW)], x_v)
    for j in range(2):
        pltpu.async_copy(
            emb_hbm.at[idx_v.at[j]], rows_v.at[pl.ds(j * 128, 128)], sem
        ).wait()

    def body(r, acc):
        d0 = rows_v[r, pl.ds(0, 16)] - x_v[r, pl.ds(0, 16)]
        d1 = rows_v[r, pl.ds(16, 16)] - x_v[r, pl.ds(16, 16)]
        st_v[r, pl.ds(0, 16)] = x_v[r, pl.ds(0, 16)] + d0
        st_v[r, pl.ds(16, 16)] = x_v[r, pl.ds(16, 16)] + d1
        return acc + d0 * d0 + d1 * d1

    acc = lax.fori_loop(0, BW, body, jnp.zeros((16,), jnp.float32))
    acc_v[...] = acc
    pltpu.sync_copy(st_v, q_out.at[pl.ds(base, BW)])
    pltpu.sync_copy(acc_v, loss_out.at[wid])


def _sc_gather(idx_sc, flat, emb_pad):
    return pl.kernel(
        _sc_gather_body,
        mesh=plsc.VectorSubcoreMesh(core_axis_name="c", subcore_axis_name="s"),
        out_type=[
            jax.ShapeDtypeStruct((N, D), jnp.float32),
            jax.ShapeDtypeStruct((NW, 16), jnp.float32),
        ],
        scratch_types=[
            pltpu.VMEM((2, 128), jnp.int32),
            pltpu.VMEM((BW, 128), jnp.float32),
            pltpu.VMEM((BW, D), jnp.float32),
            pltpu.VMEM((BW, D), jnp.float32),
            pltpu.VMEM((16,), jnp.float32),
            pltpu.SemaphoreType.DMA,
        ],
    )(idx_sc, flat, emb_pad)


def kernel(inputs, emb_w):
    input_shape = inputs.shape
    flat = inputs.reshape(-1, D)
    xsq = jnp.sum(flat ** 2, axis=1, keepdims=True)
    esq = jnp.sum(emb_w ** 2, axis=1).reshape(1, K)
    et = emb_w.T

    encodings, idx, ppl = _vq_tc(flat, et, xsq, esq)
    idx_sc = idx.reshape(NW, 2, 128)
    emb_pad = jnp.pad(emb_w, ((0, 0), (0, 128 - D)))
    q_st, loss_parts = _sc_gather(idx_sc, flat, emb_pad)

    e_latent = jnp.sum(loss_parts) / (N * D)
    vq_loss = COMMITMENT_COST * e_latent
    quantized_st = q_st.reshape(input_shape)
    perplexity = ppl[0, 0]
    return (vq_loss, quantized_st, perplexity, encodings)
